# Initial kernel scaffold; baseline (speedup 1.0000x reference)
import jax
import jax.numpy as jnp
from jax.experimental import pallas as pl


def kernel(x, W, b):
    raise NotImplementedError("write your pallas kernel here")


# speedup vs baseline: 1.1816x; 1.1816x over previous
import functools

import jax
import jax.numpy as jnp
from jax.experimental import pallas as pl

_E = 64
_TOP_K = 8
_TM = 1024


def _gate_kernel(x_ref, w_ref, b_ref, wout_ref, iout_ref):
    logits = jnp.dot(x_ref[...], w_ref[...], preferred_element_type=jnp.float32)
    logits = logits + b_ref[...]
    m = jnp.max(logits, axis=-1, keepdims=True)
    e = jnp.exp(logits - m)
    s = jnp.sum(e, axis=-1, keepdims=True)
    scores = e / s
    idx = jax.lax.broadcasted_iota(jnp.int32, scores.shape, 1)
    cur = scores
    ws = []
    inds = []
    for _ in range(_TOP_K):
        mk = jnp.max(cur, axis=-1, keepdims=True)
        is_max = cur == mk
        ik = jnp.min(jnp.where(is_max, idx, _E), axis=-1, keepdims=True)
        ws.append(mk)
        inds.append(ik)
        cur = jnp.where(idx == ik, -1.0, cur)
    wout_ref[...] = jnp.concatenate(ws, axis=-1)
    iout_ref[...] = jnp.concatenate(inds, axis=-1)


@jax.jit
def kernel(x, W, b):
    B, S, D = x.shape
    T = B * S
    x2 = x.reshape(T, D)
    grid = (T // _TM,)
    weights, indices = pl.pallas_call(
        _gate_kernel,
        grid=grid,
        in_specs=[
            pl.BlockSpec((_TM, D), lambda i: (i, 0)),
            pl.BlockSpec((D, _E), lambda i: (0, 0)),
            pl.BlockSpec((_E,), lambda i: (0,)),
        ],
        out_specs=[
            pl.BlockSpec((_TM, _TOP_K), lambda i: (i, 0)),
            pl.BlockSpec((_TM, _TOP_K), lambda i: (i, 0)),
        ],
        out_shape=[
            jax.ShapeDtypeStruct((T, _TOP_K), jnp.float32),
            jax.ShapeDtypeStruct((T, _TOP_K), jnp.int32),
        ],
    )(x2, W, b)
    return weights.reshape(B, S, _TOP_K), indices.reshape(B, S, _TOP_K)


# speedup vs baseline: 2.1778x; 1.8431x over previous
import jax
import jax.numpy as jnp
from jax.experimental import pallas as pl

_E = 64
_TOP_K = 8
_TM = 1024


def _gate_kernel_t(x_ref, w_ref, b_ref, wout_ref, iout_ref):
    logits = jax.lax.dot_general(
        w_ref[...], x_ref[...],
        dimension_numbers=(((0,), (1,)), ((), ())),
        preferred_element_type=jnp.float32,
    )
    logits = logits + b_ref[...]
    m = jnp.max(logits, axis=0, keepdims=True)
    e = jnp.exp(logits - m)
    s = jnp.sum(e, axis=0, keepdims=True)
    scores = e / s
    idx = jax.lax.broadcasted_iota(jnp.int32, scores.shape, 0)
    cur = scores
    ws = []
    inds = []
    for _ in range(_TOP_K):
        mk = jnp.max(cur, axis=0, keepdims=True)
        is_max = cur == mk
        ik = jnp.min(jnp.where(is_max, idx, _E), axis=0, keepdims=True)
        ws.append(mk)
        inds.append(ik)
        cur = jnp.where(idx == ik, -1.0, cur)
    wout_ref[...] = jnp.concatenate(ws, axis=0)
    iout_ref[...] = jnp.concatenate(inds, axis=0)


@jax.jit
def kernel(x, W, b):
    B, S, D = x.shape
    T = B * S
    x2 = x.reshape(T, D)
    b2 = b.reshape(_E, 1)
    grid = (T // _TM,)
    weights_t, indices_t = pl.pallas_call(
        _gate_kernel_t,
        grid=grid,
        in_specs=[
            pl.BlockSpec((_TM, D), lambda i: (i, 0)),
            pl.BlockSpec((D, _E), lambda i: (0, 0)),
            pl.BlockSpec((_E, 1), lambda i: (0, 0)),
        ],
        out_specs=[
            pl.BlockSpec((_TOP_K, _TM), lambda i: (0, i)),
            pl.BlockSpec((_TOP_K, _TM), lambda i: (0, i)),
        ],
        out_shape=[
            jax.ShapeDtypeStruct((_TOP_K, T), jnp.float32),
            jax.ShapeDtypeStruct((_TOP_K, T), jnp.int32),
        ],
    )(x2, W, b2)
    weights = weights_t.T.reshape(B, S, _TOP_K)
    indices = indices_t.T.reshape(B, S, _TOP_K)
    return weights, indices


# speedup vs baseline: 2.3230x; 1.0667x over previous
import jax
import jax.numpy as jnp
from jax.experimental import pallas as pl

_E = 64
_TOP_K = 8
_TM = 2048


def _gate_kernel_t(x_ref, w_ref, b_ref, wout_ref, iout_ref):
    logits = jax.lax.dot_general(
        w_ref[...], x_ref[...],
        dimension_numbers=(((0,), (1,)), ((), ())),
        preferred_element_type=jnp.float32,
    )
    logits = logits + b_ref[...]
    m = jnp.max(logits, axis=0, keepdims=True)
    e = jnp.exp(logits - m)
    s = jnp.sum(e, axis=0, keepdims=True)
    scores = e / s
    idx = jax.lax.broadcasted_iota(jnp.int32, scores.shape, 0)
    cur = scores
    ws = []
    inds = []
    for _ in range(_TOP_K):
        mk = jnp.max(cur, axis=0, keepdims=True)
        is_max = cur == mk
        ik = jnp.min(jnp.where(is_max, idx, _E), axis=0, keepdims=True)
        ws.append(mk)
        inds.append(ik)
        cur = jnp.where(idx == ik, -1.0, cur)
    wout_ref[...] = jnp.concatenate(ws, axis=0)
    iout_ref[...] = jnp.concatenate(inds, axis=0)


@jax.jit
def kernel(x, W, b):
    B, S, D = x.shape
    T = B * S
    x2 = x.reshape(T, D)
    b2 = b.reshape(_E, 1)
    grid = (T // _TM,)
    weights_t, indices_t = pl.pallas_call(
        _gate_kernel_t,
        grid=grid,
        in_specs=[
            pl.BlockSpec((_TM, D), lambda i: (i, 0)),
            pl.BlockSpec((D, _E), lambda i: (0, 0)),
            pl.BlockSpec((_E, 1), lambda i: (0, 0)),
        ],
        out_specs=[
            pl.BlockSpec((_TOP_K, _TM), lambda i: (0, i)),
            pl.BlockSpec((_TOP_K, _TM), lambda i: (0, i)),
        ],
        out_shape=[
            jax.ShapeDtypeStruct((_TOP_K, T), jnp.float32),
            jax.ShapeDtypeStruct((_TOP_K, T), jnp.int32),
        ],
    )(x2, W, b2)
    weights = weights_t.T.reshape(B, S, _TOP_K)
    indices = indices_t.T.reshape(B, S, _TOP_K)
    return weights, indices
